# baseline (speedup 1.0000x reference)
import functools

import jax
import jax.numpy as jnp
from jax import lax
from jax.experimental import pallas as pl
from jax.experimental.pallas import tpu as pltpu
from jax.experimental.pallas import tpu_sc as plsc

B, S, D = 16384, 100, 64
NC, NS, L = 2, 16, 16
NW = NC * NS
DG = 16
CW = 1024
NLG = B // CW
NDG = D // DG
NQ = S * NDG * NLG
QPW = NQ // NW
NBUF = 4
PF = 2


def _make_sc_add():
    mesh = plsc.VectorSubcoreMesh(
        core_axis_name="c", subcore_axis_name="s", num_cores=NC, num_subcores=NS
    )

    @functools.partial(
        pl.kernel,
        mesh=mesh,
        out_type=jax.ShapeDtypeStruct((S, D, B), jnp.float32),
        scratch_types=[
            pltpu.VMEM((S * D,), jnp.float32),
            pltpu.VMEM((DG, CW), jnp.float32),
            pltpu.VMEM((DG, CW), jnp.float32),
            pltpu.VMEM((DG, CW), jnp.float32),
            pltpu.VMEM((DG, CW), jnp.float32),
            pltpu.SemaphoreType.DMA,
            pltpu.SemaphoreType.DMA,
            pltpu.SemaphoreType.DMA,
            pltpu.SemaphoreType.DMA,
            pltpu.SemaphoreType.DMA,
            pltpu.SemaphoreType.DMA,
            pltpu.SemaphoreType.DMA,
            pltpu.SemaphoreType.DMA,
        ],
    )
    def k(
        tok_hbm, emb_hbm, out_hbm,
        emb_v, b0, b1, b2, b3,
        is0, is1, is2, is3, os0, os1, os2, os3,
    ):
        bufs = [b0, b1, b2, b3]
        isems = [is0, is1, is2, is3]
        osems = [os0, os1, os2, os3]
        wid = lax.axis_index("s") * NC + lax.axis_index("c")
        q0 = wid * QPW
        pltpu.sync_copy(emb_hbm, emb_v)

        def q_slices(q):
            s = q // (NDG * NLG)
            rem = q % (NDG * NLG)
            dg = rem // NLG
            lg = rem % NLG
            return s, dg * DG, lg * CW

        def in_copy(q, b):
            s, d0, c0 = q_slices(q)
            return pltpu.make_async_copy(
                tok_hbm.at[s, pl.ds(d0, DG), pl.ds(c0, CW)], bufs[b], isems[b]
            )

        def out_copy(q, b):
            s, d0, c0 = q_slices(q)
            return pltpu.make_async_copy(
                bufs[b], out_hbm.at[s, pl.ds(d0, DG), pl.ds(c0, CW)], osems[b]
            )

        for b in range(PF):
            in_copy(q0 + b, b).start()

        def step(m, carry):
            for b in range(NBUF):
                g = m * NBUF + b
                q = q0 + g
                in_copy(q, b).wait()

                s, d0, _ = q_slices(q)
                e0 = pl.multiple_of(s * D + d0, L)
                ev = emb_v[pl.ds(e0, L)]
                evs = [
                    jnp.full((L,), ev[r], dtype=jnp.float32)
                    for r in range(DG)
                ]

                @plsc.parallel_loop(0, CW // L, unroll=8)
                def _(i):
                    col = pl.multiple_of(i * L, L)
                    for r in range(DG):
                        bufs[b][r, pl.ds(col, L)] = (
                            bufs[b][r, pl.ds(col, L)] + evs[r]
                        )

                out_copy(q, b).start()

                @pl.when(g >= PF)
                def _():
                    out_copy(q - PF, (b - PF) % NBUF).wait()

                @pl.when(g + PF < QPW)
                def _():
                    in_copy(q + PF, (b + PF) % NBUF).start()

            return carry

        lax.fori_loop(0, QPW // NBUF, step, 0)

        for b in range(PF):
            g = QPW - PF + b
            out_copy(q0 + g, g % NBUF).wait()

    return k


_sc_add = _make_sc_add()


def kernel(tokens, id_embedding):
    tok_t = jnp.transpose(tokens, (1, 2, 0))
    emb = id_embedding.reshape(S * D)
    out_t = _sc_add(tok_t, emb)
    return jnp.transpose(out_t, (2, 0, 1))


# speedup vs baseline: 1.0122x; 1.0122x over previous
import functools

import jax
import jax.numpy as jnp
from jax import lax
from jax.experimental import pallas as pl
from jax.experimental.pallas import tpu as pltpu
from jax.experimental.pallas import tpu_sc as plsc

B, S, D = 16384, 100, 64
NC, NS, L = 2, 16, 16
NW = NC * NS
NQ = S * D
QPW = NQ // NW
NBUF = 4
PF = 2


def _make_sc_add():
    mesh = plsc.VectorSubcoreMesh(
        core_axis_name="c", subcore_axis_name="s", num_cores=NC, num_subcores=NS
    )

    @functools.partial(
        pl.kernel,
        mesh=mesh,
        out_type=jax.ShapeDtypeStruct((S, D, B), jnp.float32),
        scratch_types=[
            pltpu.VMEM((QPW, L), jnp.float32),
            pltpu.VMEM((1, B), jnp.float32),
            pltpu.VMEM((1, B), jnp.float32),
            pltpu.VMEM((1, B), jnp.float32),
            pltpu.VMEM((1, B), jnp.float32),
            pltpu.SemaphoreType.DMA,
            pltpu.SemaphoreType.DMA,
            pltpu.SemaphoreType.DMA,
            pltpu.SemaphoreType.DMA,
            pltpu.SemaphoreType.DMA,
            pltpu.SemaphoreType.DMA,
            pltpu.SemaphoreType.DMA,
            pltpu.SemaphoreType.DMA,
        ],
    )
    def k(
        tok_hbm, emb16_hbm, out_hbm,
        emb_v, b0, b1, b2, b3,
        is0, is1, is2, is3, os0, os1, os2, os3,
    ):
        bufs = [b0, b1, b2, b3]
        isems = [is0, is1, is2, is3]
        osems = [os0, os1, os2, os3]
        wid = lax.axis_index("s") * NC + lax.axis_index("c")
        q0 = wid * QPW
        pltpu.sync_copy(emb16_hbm.at[pl.ds(q0, QPW), :], emb_v)

        def in_copy(q, b):
            return pltpu.make_async_copy(
                tok_hbm.at[q // D, pl.ds(q % D, 1), :], bufs[b], isems[b]
            )

        def out_copy(q, b):
            return pltpu.make_async_copy(
                bufs[b], out_hbm.at[q // D, pl.ds(q % D, 1), :], osems[b]
            )

        for b in range(PF):
            in_copy(q0 + b, b).start()

        def step(m, carry):
            for b in range(NBUF):
                g = m * NBUF + b
                q = q0 + g
                in_copy(q, b).wait()

                ev = emb_v[g, :]

                @plsc.parallel_loop(0, B // L, unroll=8)
                def _(i):
                    col = pl.multiple_of(i * L, L)
                    bufs[b][0, pl.ds(col, L)] = bufs[b][0, pl.ds(col, L)] + ev

                out_copy(q, b).start()

                @pl.when(g >= PF)
                def _():
                    out_copy(q - PF, (b - PF) % NBUF).wait()

                @pl.when(g + PF < QPW)
                def _():
                    in_copy(q + PF, (b + PF) % NBUF).start()

            return carry

        lax.fori_loop(0, QPW // NBUF, step, 0)

        for b in range(PF):
            g = QPW - PF + b
            out_copy(q0 + g, g % NBUF).wait()

    return k


_sc_add = _make_sc_add()


def kernel(tokens, id_embedding):
    tok_t = jnp.transpose(tokens, (1, 2, 0))
    emb16 = jnp.broadcast_to(
        id_embedding.reshape(S * D, 1), (S * D, L)
    )
    out_t = _sc_add(tok_t, emb16)
    return jnp.transpose(out_t, (2, 0, 1))


# speedup vs baseline: 1.0127x; 1.0005x over previous
import functools

import jax
import jax.numpy as jnp
from jax import lax
from jax.experimental import pallas as pl
from jax.experimental.pallas import tpu as pltpu
from jax.experimental.pallas import tpu_sc as plsc

B, S, D = 16384, 100, 64
NC, NS, L = 2, 16, 16
NW = NC * NS
NQ = S * D
QPW = NQ // NW
CPR = 2
NCH = QPW // CPR
NBUF = 3
MAIN = (NCH // NBUF) * NBUF


def _make_sc_add():
    mesh = plsc.VectorSubcoreMesh(
        core_axis_name="c", subcore_axis_name="s", num_cores=NC, num_subcores=NS
    )

    @functools.partial(
        pl.kernel,
        mesh=mesh,
        out_type=jax.ShapeDtypeStruct((S, D, B), jnp.float32),
        scratch_types=[
            pltpu.VMEM((QPW, L), jnp.float32),
            pltpu.VMEM((CPR, B), jnp.float32),
            pltpu.VMEM((CPR, B), jnp.float32),
            pltpu.VMEM((CPR, B), jnp.float32),
            pltpu.SemaphoreType.DMA,
            pltpu.SemaphoreType.DMA,
            pltpu.SemaphoreType.DMA,
            pltpu.SemaphoreType.DMA,
            pltpu.SemaphoreType.DMA,
            pltpu.SemaphoreType.DMA,
        ],
    )
    def k(
        tok_hbm, emb16_hbm, out_hbm,
        emb_v, b0, b1, b2,
        is0, is1, is2, os0, os1, os2,
    ):
        bufs = [b0, b1, b2]
        isems = [is0, is1, is2]
        osems = [os0, os1, os2]
        wid = lax.axis_index("s") * NC + lax.axis_index("c")
        q0 = wid * QPW
        pltpu.sync_copy(emb16_hbm.at[pl.ds(q0, QPW), :], emb_v)

        def in_copy(g, b):
            r = q0 + g * CPR
            return pltpu.make_async_copy(
                tok_hbm.at[r // D, pl.ds(r % D, CPR), :], bufs[b], isems[b]
            )

        def out_copy(g, b):
            r = q0 + g * CPR
            return pltpu.make_async_copy(
                bufs[b], out_hbm.at[r // D, pl.ds(r % D, CPR), :], osems[b]
            )

        def process(g, b):
            in_copy(g, b).wait()

            @pl.when(g >= NBUF - 1)
            def _():
                out_copy(g - (NBUF - 1), (b + 1) % NBUF).wait()

            @pl.when(g + 1 < NCH)
            def _():
                in_copy(g + 1, (b + 1) % NBUF).start()

            for kk in range(CPR):
                ev = emb_v[g * CPR + kk, :]

                @plsc.parallel_loop(0, B // L, unroll=8)
                def _(i):
                    col = pl.multiple_of(i * L, L)
                    bufs[b][kk, pl.ds(col, L)] = bufs[b][kk, pl.ds(col, L)] + ev

            out_copy(g, b).start()

        in_copy(0, 0).start()

        def step(m, carry):
            for b in range(NBUF):
                process(m * NBUF + b, b)
            return carry

        lax.fori_loop(0, MAIN // NBUF, step, 0)

        for g in range(MAIN, NCH):
            process(g, g % NBUF)

        for g in range(NCH - (NBUF - 1), NCH):
            out_copy(g, g % NBUF).wait()

    return k


_sc_add = _make_sc_add()


def kernel(tokens, id_embedding):
    tok_t = jnp.transpose(tokens, (1, 2, 0))
    emb16 = jnp.broadcast_to(
        id_embedding.reshape(S * D, 1), (S * D, L)
    )
    out_t = _sc_add(tok_t, emb16)
    return jnp.transpose(out_t, (2, 0, 1))
